# Initial kernel scaffold; baseline (speedup 1.0000x reference)
import functools

import jax
import jax.numpy as jnp
from jax import lax
from jax.experimental import pallas as pl
from jax.experimental.pallas import tpu as pltpu
from jax.experimental.pallas import tpu_sc as plsc


def _make_emb_kernel(B, D, NW, b_per_w, CH):
    NCH = b_per_w // CH
    mesh = plsc.VectorSubcoreMesh(core_axis_name="c", subcore_axis_name="s")
    nc = mesh.num_cores

    @functools.partial(
        pl.kernel,
        out_type=jax.ShapeDtypeStruct((B, D), jnp.float32),
        mesh=mesh,
        scratch_types=[
            pltpu.VMEM((b_per_w,), jnp.int32),
            pltpu.VMEM((CH, D), jnp.float32),
            pltpu.SemaphoreType.DMA,
        ],
    )
    def emb(idx_hbm, table_hbm, out_hbm, idx_v, rows_v, sem):
        wid = lax.axis_index("s") * nc + lax.axis_index("c")
        base = wid * b_per_w
        pltpu.sync_copy(idx_hbm.at[pl.ds(base, b_per_w)], idx_v)

        def body(c, carry):
            off = c * CH
            pltpu.async_copy(
                table_hbm.at[idx_v.at[pl.ds(off, CH)]], rows_v, sem
            ).wait()
            pltpu.sync_copy(rows_v, out_hbm.at[pl.ds(base + off, CH)])
            return carry

        lax.fori_loop(0, NCH, body, 0)

    return emb


def kernel(text_input_ids, embedding_table):
    D = embedding_table.shape[1]
    idx = text_input_ids.reshape(-1).astype(jnp.int32)
    B = idx.shape[0]
    NW = 32
    b_per_w = B // NW
    CH = 128
    emb = _make_emb_kernel(B, D, NW, b_per_w, CH)
    out = emb(idx, embedding_table)
    return out.reshape(text_input_ids.shape + (D,))


# speedup vs baseline: 1.6845x; 1.6845x over previous
import functools

import jax
import jax.numpy as jnp
from jax import lax
from jax.experimental import pallas as pl
from jax.experimental.pallas import tpu as pltpu
from jax.experimental.pallas import tpu_sc as plsc


def _make_emb_kernel(B, D, NW, b_per_w, CH):
    NCH = b_per_w // CH
    mesh = plsc.VectorSubcoreMesh(core_axis_name="c", subcore_axis_name="s")
    nc = mesh.num_cores

    @functools.partial(
        pl.kernel,
        out_type=jax.ShapeDtypeStruct((B, D), jnp.float32),
        mesh=mesh,
        scratch_types=[
            pltpu.VMEM((b_per_w,), jnp.int32),
            pltpu.VMEM((CH, D), jnp.float32),
            pltpu.SemaphoreType.DMA,
        ],
        compiler_params=pltpu.CompilerParams(use_tc_tiling_on_sc=False),
    )
    def emb(idx_hbm, table_hbm, out_hbm, idx_v, rows_v, sem):
        wid = lax.axis_index("s") * nc + lax.axis_index("c")
        base = wid * b_per_w
        pltpu.sync_copy(idx_hbm.at[pl.ds(base, b_per_w)], idx_v)

        def body(c, carry):
            off = c * CH
            pltpu.async_copy(
                table_hbm.at[idx_v.at[pl.ds(off, CH)]], rows_v, sem
            ).wait()
            pltpu.sync_copy(rows_v, out_hbm.at[pl.ds(base + off, CH)])
            return carry

        lax.fori_loop(0, NCH, body, 0)

    return emb


def kernel(text_input_ids, embedding_table):
    D = embedding_table.shape[1]
    idx = text_input_ids.reshape(-1).astype(jnp.int32)
    B = idx.shape[0]
    NW = 32
    b_per_w = B // NW
    CH = 128
    emb = _make_emb_kernel(B, D, NW, b_per_w, CH)
    out = emb(idx, embedding_table)
    return out.reshape(text_input_ids.shape + (D,))


# speedup vs baseline: 1.8700x; 1.1101x over previous
import functools

import jax
import jax.numpy as jnp
from jax import lax
from jax.experimental import pallas as pl
from jax.experimental.pallas import tpu as pltpu
from jax.experimental.pallas import tpu_sc as plsc


def _make_emb_kernel(B, D, b_per_w, CH, NBUF):
    NCH = b_per_w // CH
    LAPS = NCH // NBUF
    assert NCH % NBUF == 0 and LAPS >= 2
    mesh = plsc.VectorSubcoreMesh(core_axis_name="c", subcore_axis_name="s")
    nc = mesh.num_cores

    scratch = [pltpu.VMEM((b_per_w,), jnp.int32)]
    scratch += [pltpu.VMEM((CH, D), jnp.float32) for _ in range(NBUF)]
    scratch += [pltpu.SemaphoreType.DMA for _ in range(2 * NBUF)]

    @functools.partial(
        pl.kernel,
        out_type=jax.ShapeDtypeStruct((B, D), jnp.float32),
        mesh=mesh,
        scratch_types=scratch,
        compiler_params=pltpu.CompilerParams(use_tc_tiling_on_sc=False),
    )
    def emb(idx_hbm, table_hbm, out_hbm, idx_v, *bufs_sems):
        rows = bufs_sems[:NBUF]
        gsem = bufs_sems[NBUF : 2 * NBUF]
        ssem = bufs_sems[2 * NBUF :]
        wid = lax.axis_index("s") * nc + lax.axis_index("c")
        base = wid * b_per_w
        pltpu.sync_copy(idx_hbm.at[pl.ds(base, b_per_w)], idx_v)

        def gather(c, b):
            return pltpu.async_copy(
                table_hbm.at[idx_v.at[pl.ds(c * CH, CH)]], rows[b], gsem[b]
            )

        def scatter(c, b):
            return pltpu.async_copy(
                rows[b], out_hbm.at[pl.ds(base + c * CH, CH)], ssem[b]
            )

        def scatter_wait(b):
            pltpu.make_async_copy(
                rows[b], out_hbm.at[pl.ds(base, CH)], ssem[b]
            ).wait()

        handles = [gather(b, b) for b in range(NBUF)]
        for b in range(NBUF):
            handles[b].wait()
            scatter(b, b)

        def lap(L, carry):
            c0 = L * NBUF
            hs = []
            for b in range(NBUF):
                scatter_wait(b)
                hs.append(gather(c0 + b, b))
            for b in range(NBUF):
                hs[b].wait()
                scatter(c0 + b, b)
            return carry

        lax.fori_loop(1, LAPS, lap, 0)

        for b in range(NBUF):
            scatter_wait(b)

    return emb


def kernel(text_input_ids, embedding_table):
    D = embedding_table.shape[1]
    idx = text_input_ids.reshape(-1).astype(jnp.int32)
    B = idx.shape[0]
    NW = 32
    b_per_w = B // NW
    CH = 320
    NBUF = 4
    emb = _make_emb_kernel(B, D, b_per_w, CH, NBUF)
    out = emb(idx, embedding_table)
    return out.reshape(text_input_ids.shape + (D,))
